# baseline (speedup 1.0000x reference)
import functools

import jax
import jax.numpy as jnp
import numpy as np
from jax import lax
from jax.experimental import pallas as pl
from jax.experimental.pallas import tpu as pltpu
from jax.experimental.pallas import tpu_sc as plsc

B = 64
D = 8192
RATE = 819
CHUNKS = (RATE + 15) // 16
IDXP = CHUNKS * 16
TINY = np.float32(np.finfo(np.float32).tiny)


def _rotl(x, r):
    return (x << jnp.uint32(r)) | (x >> jnp.uint32(32 - r))


def _threefry_bits(c1):
    k0 = jnp.uint32(0)
    k1 = jnp.uint32(42)
    k2 = jnp.uint32(0 ^ 42 ^ 0x1BD11BDA)
    r1 = (13, 15, 26, 6)
    r2 = (17, 29, 16, 24)

    def rounds(x0, x1, rots):
        for r in rots:
            x0 = x0 + x1
            x1 = _rotl(x1, r)
            x1 = x1 ^ x0
        return x0, x1

    x0 = jnp.zeros_like(c1) + k0
    x1 = c1 + k1
    x0, x1 = rounds(x0, x1, r1)
    x0 = x0 + k1
    x1 = x1 + (k2 + jnp.uint32(1))
    x0, x1 = rounds(x0, x1, r2)
    x0 = x0 + k2
    x1 = x1 + (k0 + jnp.uint32(2))
    x0, x1 = rounds(x0, x1, r1)
    x0 = x0 + k0
    x1 = x1 + (k1 + jnp.uint32(3))
    x0, x1 = rounds(x0, x1, r2)
    x0 = x0 + k1
    x1 = x1 + (k2 + jnp.uint32(4))
    x0, x1 = rounds(x0, x1, r1)
    x0 = x0 + k2
    x1 = x1 + (k0 + jnp.uint32(5))
    return x0 ^ x1


CHUNK = 256
NCHUNK = D // CHUNK
DRAWS = 3
NSTEP = RATE // DRAWS


def _threefry_bits_from_x1(v):
    k1 = jnp.uint32(42)
    k2 = jnp.uint32(0 ^ 42 ^ 0x1BD11BDA)
    k0 = jnp.uint32(0)
    r1 = (13, 15, 26, 6)
    r2 = (17, 29, 16, 24)

    def rounds(x0, x1, rots):
        for r in rots:
            x0 = x0 + x1
            x1 = _rotl(x1, r)
            x1 = x1 ^ x0
        return x0, x1

    x0 = v
    x1 = _rotl(v, r1[0]) ^ x0
    x0, x1 = rounds(x0, x1, r1[1:])
    x0 = x0 + k1
    x1 = x1 + (k2 + jnp.uint32(1))
    x0, x1 = rounds(x0, x1, r2)
    x0 = x0 + k2
    x1 = x1 + (k0 + jnp.uint32(2))
    x0, x1 = rounds(x0, x1, r1)
    x0 = x0 + k0
    x1 = x1 + (k1 + jnp.uint32(3))
    x0, x1 = rounds(x0, x1, r2)
    x0 = x0 + k1
    x1 = x1 + (k2 + jnp.uint32(4))
    x0, x1 = rounds(x0, x1, r1)
    x0 = x0 + k2
    x1 = x1 + (k0 + jnp.uint32(5))
    return x0 ^ x1


def _sample_body(logits_ref, x_ref, p_ref, idx_ref, val_ref):
    step = pl.program_id(0)
    row = lax.broadcasted_iota(jnp.uint32, (B, CHUNK), 0)
    lane = lax.broadcasted_iota(jnp.uint32, (B, CHUNK), 1)
    kbase0 = row * jnp.uint32(D) + lane + jnp.uint32(42)
    col0 = lax.broadcasted_iota(jnp.int32, (B, CHUNK), 1)

    for dr in range(DRAWS):
        r = step * DRAWS + dr
        base = r.astype(jnp.uint32) * jnp.uint32(B * D)
        kbase = kbase0 + base

        def chunk_body(k, carry):
            acc_m, acc_k = carry
            v = kbase + (k.astype(jnp.uint32) * jnp.uint32(CHUNK))
            bits = _threefry_bits_from_x1(v)
            f = lax.bitcast_convert_type(
                (bits >> jnp.uint32(9)) | jnp.uint32(0x3F800000), jnp.float32)
            f = f - np.float32(1.0)
            u = jnp.maximum(f * (np.float32(1.0) - TINY) + TINY, TINY)
            g = -jnp.log(-jnp.log(u))
            s = g + logits_ref[:, pl.ds(k * CHUNK, CHUNK)]
            upd = s > acc_m
            acc_m = jnp.where(upd, s, acc_m)
            acc_k = jnp.where(upd, k, acc_k)
            return acc_m, acc_k

        acc_m0 = jnp.full((B, CHUNK), -jnp.inf, jnp.float32)
        acc_k0 = jnp.zeros((B, CHUNK), jnp.int32)
        acc_m, acc_k = lax.fori_loop(0, NCHUNK, chunk_body, (acc_m0, acc_k0))
        m = jnp.max(acc_m, axis=1, keepdims=True)
        cols = acc_k * CHUNK + col0
        idx = jnp.min(jnp.where(acc_m == m, cols, D), axis=1)
        idx_ref[dr] = idx.reshape(1, B).astype(jnp.int32)

    @pl.when(step == 0)
    def _():
        p = p_ref[...]
        e = jnp.exp(np.float32(RATE) * jnp.log(np.float32(1.0) - p))
        pre = np.float32(1.0) / (np.float32(1.0) - e)
        pre = jnp.where(jnp.isinf(pre), np.float32(1.0) / (np.float32(RATE) * p), pre)
        val_ref[...] = x_ref[...] * pre


_sample_call = pl.pallas_call(
    _sample_body,
    grid=(NSTEP,),
    in_specs=[
        pl.BlockSpec((B, D), lambda r: (0, 0)),
        pl.BlockSpec((B, D), lambda r: (0, 0)),
        pl.BlockSpec((B, D), lambda r: (0, 0)),
    ],
    out_specs=[
        pl.BlockSpec((DRAWS, 1, B), lambda r: (r, 0, 0)),
        pl.BlockSpec((B, D), lambda r: (0, 0)),
    ],
    out_shape=[
        jax.ShapeDtypeStruct((RATE, 1, B), jnp.int32),
        jax.ShapeDtypeStruct((B, D), jnp.float32),
    ],
    compiler_params=pltpu.CompilerParams(
        dimension_semantics=("parallel",),
    ),
)


def _make_scatter_kernel():
    info = plsc.get_sparse_core_info()
    nw = info.num_cores * info.num_subcores
    rows_per_w = B // nw

    @functools.partial(
        pl.kernel,
        mesh=plsc.VectorSubcoreMesh(core_axis_name="c", subcore_axis_name="s"),
        out_type=jax.ShapeDtypeStruct((B, D), jnp.float32),
        scratch_types=[
            pltpu.VMEM((IDXP,), jnp.int32),
            pltpu.VMEM((D,), jnp.float32),
            pltpu.VMEM((D,), jnp.float32),
        ],
        compiler_params=pltpu.CompilerParams(needs_layout_passes=False),
    )
    def _scatter_kernel(idx_hbm, val_hbm, out_hbm, idx_v, val_v, out_v):
        wid = lax.axis_index("s") * info.num_cores + lax.axis_index("c")
        zero = jnp.zeros((16,), jnp.float32)
        for t in range(rows_per_w):
            b = wid * rows_per_w + t
            pltpu.sync_copy(idx_hbm.at[b], idx_v)
            pltpu.sync_copy(val_hbm.at[b], val_v)

            def zbody(j, _):
                out_v[pl.ds(j * 16, 16)] = zero
                return _

            lax.fori_loop(0, D // 16, zbody, None)
            for c in range(CHUNKS):
                iv = idx_v[pl.ds(c * 16, 16)]
                vals = plsc.load_gather(val_v, [iv])
                plsc.store_scatter(out_v, [iv], vals)
            pltpu.sync_copy(out_v, out_hbm.at[b])

    return _scatter_kernel


def kernel(x):
    ha = jnp.abs(x)
    p = ha / (jnp.sum(ha, axis=1, keepdims=True) + 1e-10)
    logits = jnp.where(p > 0, jnp.log(p), -jnp.inf)
    idx, val = _sample_call(logits, x, p)
    idx_t = idx.reshape(RATE, B).T
    idx_pad = jnp.concatenate(
        [idx_t, jnp.broadcast_to(idx_t[:, -1:], (B, IDXP - RATE))], axis=1)
    return _make_scatter_kernel()(idx_pad, val)


# speedup vs baseline: 1.0312x; 1.0312x over previous
import functools

import jax
import jax.numpy as jnp
import numpy as np
from jax import lax
from jax.experimental import pallas as pl
from jax.experimental.pallas import tpu as pltpu
from jax.experimental.pallas import tpu_sc as plsc

B = 64
D = 8192
RATE = 819
CHUNKS = (RATE + 15) // 16
IDXP = CHUNKS * 16
TINY = np.float32(np.finfo(np.float32).tiny)


def _rotl(x, r):
    return (x << jnp.uint32(r)) | (x >> jnp.uint32(32 - r))


def _threefry_bits(c1):
    k0 = jnp.uint32(0)
    k1 = jnp.uint32(42)
    k2 = jnp.uint32(0 ^ 42 ^ 0x1BD11BDA)
    r1 = (13, 15, 26, 6)
    r2 = (17, 29, 16, 24)

    def rounds(x0, x1, rots):
        for r in rots:
            x0 = x0 + x1
            x1 = _rotl(x1, r)
            x1 = x1 ^ x0
        return x0, x1

    x0 = jnp.zeros_like(c1) + k0
    x1 = c1 + k1
    x0, x1 = rounds(x0, x1, r1)
    x0 = x0 + k1
    x1 = x1 + (k2 + jnp.uint32(1))
    x0, x1 = rounds(x0, x1, r2)
    x0 = x0 + k2
    x1 = x1 + (k0 + jnp.uint32(2))
    x0, x1 = rounds(x0, x1, r1)
    x0 = x0 + k0
    x1 = x1 + (k1 + jnp.uint32(3))
    x0, x1 = rounds(x0, x1, r2)
    x0 = x0 + k1
    x1 = x1 + (k2 + jnp.uint32(4))
    x0, x1 = rounds(x0, x1, r1)
    x0 = x0 + k2
    x1 = x1 + (k0 + jnp.uint32(5))
    return x0 ^ x1


CHUNK = 256
NCHUNK = D // CHUNK
DRAWS = 9
NSTEP = RATE // DRAWS


def _threefry_bits_from_x1(v):
    k1 = jnp.uint32(42)
    k2 = jnp.uint32(0 ^ 42 ^ 0x1BD11BDA)
    k0 = jnp.uint32(0)
    r1 = (13, 15, 26, 6)
    r2 = (17, 29, 16, 24)

    def rounds(x0, x1, rots):
        for r in rots:
            x0 = x0 + x1
            x1 = _rotl(x1, r)
            x1 = x1 ^ x0
        return x0, x1

    x0 = v
    x1 = _rotl(v, r1[0]) ^ x0
    x0, x1 = rounds(x0, x1, r1[1:])
    x0 = x0 + k1
    x1 = x1 + (k2 + jnp.uint32(1))
    x0, x1 = rounds(x0, x1, r2)
    x0 = x0 + k2
    x1 = x1 + (k0 + jnp.uint32(2))
    x0, x1 = rounds(x0, x1, r1)
    x0 = x0 + k0
    x1 = x1 + (k1 + jnp.uint32(3))
    x0, x1 = rounds(x0, x1, r2)
    x0 = x0 + k1
    x1 = x1 + (k2 + jnp.uint32(4))
    x0, x1 = rounds(x0, x1, r1)
    x0 = x0 + k2
    x1 = x1 + (k0 + jnp.uint32(5))
    return x0 ^ x1


def _sample_body(logits_ref, x_ref, p_ref, idx_ref, val_ref):
    step = pl.program_id(0)
    row = lax.broadcasted_iota(jnp.uint32, (B, CHUNK), 0)
    lane = lax.broadcasted_iota(jnp.uint32, (B, CHUNK), 1)
    kbase0 = row * jnp.uint32(D) + lane + jnp.uint32(42)
    col0 = lax.broadcasted_iota(jnp.int32, (B, CHUNK), 1)

    for dr in range(DRAWS):
        r = step * DRAWS + dr
        base = r.astype(jnp.uint32) * jnp.uint32(B * D)
        kbase = kbase0 + base

        def chunk_body(k, carry):
            acc_m, acc_i = carry
            v = kbase + (k.astype(jnp.uint32) * jnp.uint32(CHUNK))
            bits = _threefry_bits_from_x1(v)
            f = lax.bitcast_convert_type(
                (bits >> jnp.uint32(9)) | jnp.uint32(0x3F800000), jnp.float32)
            f = f - np.float32(1.0)
            u = jnp.maximum(f * (np.float32(1.0) - TINY) + TINY, TINY)
            g = -jnp.log(-jnp.log(u))
            s = g + logits_ref[:, pl.ds(k * CHUNK, CHUNK)]
            upd = s > acc_m
            acc_m = jnp.where(upd, s, acc_m)
            acc_i = jnp.where(upd, col0 + k * CHUNK, acc_i)
            return acc_m, acc_i

        acc_m0 = jnp.full((B, CHUNK), -jnp.inf, jnp.float32)
        acc_i0 = jnp.zeros((B, CHUNK), jnp.int32)
        acc_m, acc_i = lax.fori_loop(0, NCHUNK, chunk_body, (acc_m0, acc_i0))
        m = jnp.max(acc_m, axis=1, keepdims=True)
        idx = jnp.min(jnp.where(acc_m == m, acc_i, D), axis=1)
        idx_ref[dr] = idx.reshape(1, B).astype(jnp.int32)

    @pl.when(step == 0)
    def _():
        p = p_ref[...]
        e = jnp.exp(np.float32(RATE) * jnp.log(np.float32(1.0) - p))
        pre = np.float32(1.0) / (np.float32(1.0) - e)
        pre = jnp.where(jnp.isinf(pre), np.float32(1.0) / (np.float32(RATE) * p), pre)
        val_ref[...] = x_ref[...] * pre


_sample_call = pl.pallas_call(
    _sample_body,
    grid=(NSTEP,),
    in_specs=[
        pl.BlockSpec((B, D), lambda r: (0, 0)),
        pl.BlockSpec((B, D), lambda r: (0, 0)),
        pl.BlockSpec((B, D), lambda r: (0, 0)),
    ],
    out_specs=[
        pl.BlockSpec((DRAWS, 1, B), lambda r: (r, 0, 0)),
        pl.BlockSpec((B, D), lambda r: (0, 0)),
    ],
    out_shape=[
        jax.ShapeDtypeStruct((RATE, 1, B), jnp.int32),
        jax.ShapeDtypeStruct((B, D), jnp.float32),
    ],
    compiler_params=pltpu.CompilerParams(
        dimension_semantics=("parallel",),
    ),
)


def _make_scatter_kernel():
    info = plsc.get_sparse_core_info()
    nw = info.num_cores * info.num_subcores
    rows_per_w = B // nw

    @functools.partial(
        pl.kernel,
        mesh=plsc.VectorSubcoreMesh(core_axis_name="c", subcore_axis_name="s"),
        out_type=jax.ShapeDtypeStruct((B, D), jnp.float32),
        scratch_types=[
            pltpu.VMEM((IDXP,), jnp.int32),
            pltpu.VMEM((D,), jnp.float32),
            pltpu.VMEM((D,), jnp.float32),
        ],
        compiler_params=pltpu.CompilerParams(needs_layout_passes=False),
    )
    def _scatter_kernel(idx_hbm, val_hbm, out_hbm, idx_v, val_v, out_v):
        wid = lax.axis_index("s") * info.num_cores + lax.axis_index("c")
        zero = jnp.zeros((16,), jnp.float32)
        for t in range(rows_per_w):
            b = wid * rows_per_w + t
            pltpu.sync_copy(idx_hbm.at[b], idx_v)
            pltpu.sync_copy(val_hbm.at[b], val_v)

            def zbody(j, _):
                out_v[pl.ds(j * 16, 16)] = zero
                return _

            lax.fori_loop(0, D // 16, zbody, None)
            for c in range(CHUNKS):
                iv = idx_v[pl.ds(c * 16, 16)]
                vals = plsc.load_gather(val_v, [iv])
                plsc.store_scatter(out_v, [iv], vals)
            pltpu.sync_copy(out_v, out_hbm.at[b])

    return _scatter_kernel


def kernel(x):
    ha = jnp.abs(x)
    p = ha / (jnp.sum(ha, axis=1, keepdims=True) + 1e-10)
    logits = jnp.where(p > 0, jnp.log(p), -jnp.inf)
    idx, val = _sample_call(logits, x, p)
    idx_t = idx.reshape(RATE, B).T
    idx_pad = jnp.concatenate(
        [idx_t, jnp.broadcast_to(idx_t[:, -1:], (B, IDXP - RATE))], axis=1)
    return _make_scatter_kernel()(idx_pad, val)


# speedup vs baseline: 1.0371x; 1.0057x over previous
import functools

import jax
import jax.numpy as jnp
import numpy as np
from jax import lax
from jax.experimental import pallas as pl
from jax.experimental.pallas import tpu as pltpu
from jax.experimental.pallas import tpu_sc as plsc

B = 64
D = 8192
RATE = 819
CHUNKS = (RATE + 15) // 16
IDXP = CHUNKS * 16
TINY = np.float32(np.finfo(np.float32).tiny)


def _rotl(x, r):
    return (x << jnp.uint32(r)) | (x >> jnp.uint32(32 - r))


def _threefry_bits(c1):
    k0 = jnp.uint32(0)
    k1 = jnp.uint32(42)
    k2 = jnp.uint32(0 ^ 42 ^ 0x1BD11BDA)
    r1 = (13, 15, 26, 6)
    r2 = (17, 29, 16, 24)

    def rounds(x0, x1, rots):
        for r in rots:
            x0 = x0 + x1
            x1 = _rotl(x1, r)
            x1 = x1 ^ x0
        return x0, x1

    x0 = jnp.zeros_like(c1) + k0
    x1 = c1 + k1
    x0, x1 = rounds(x0, x1, r1)
    x0 = x0 + k1
    x1 = x1 + (k2 + jnp.uint32(1))
    x0, x1 = rounds(x0, x1, r2)
    x0 = x0 + k2
    x1 = x1 + (k0 + jnp.uint32(2))
    x0, x1 = rounds(x0, x1, r1)
    x0 = x0 + k0
    x1 = x1 + (k1 + jnp.uint32(3))
    x0, x1 = rounds(x0, x1, r2)
    x0 = x0 + k1
    x1 = x1 + (k2 + jnp.uint32(4))
    x0, x1 = rounds(x0, x1, r1)
    x0 = x0 + k2
    x1 = x1 + (k0 + jnp.uint32(5))
    return x0 ^ x1


CHUNK = 256
NCHUNK = D // CHUNK
DRAWS = 9
NSTEP = RATE // DRAWS


def _threefry_bits_from_x1(v):
    k1 = jnp.uint32(42)
    k2 = jnp.uint32(0 ^ 42 ^ 0x1BD11BDA)
    k0 = jnp.uint32(0)
    r1 = (13, 15, 26, 6)
    r2 = (17, 29, 16, 24)

    def rounds(x0, x1, rots):
        for r in rots:
            x0 = x0 + x1
            x1 = _rotl(x1, r)
            x1 = x1 ^ x0
        return x0, x1

    x0 = v
    x1 = _rotl(v, r1[0]) ^ x0
    x0, x1 = rounds(x0, x1, r1[1:])
    x0 = x0 + k1
    x1 = x1 + (k2 + jnp.uint32(1))
    x0, x1 = rounds(x0, x1, r2)
    x0 = x0 + k2
    x1 = x1 + (k0 + jnp.uint32(2))
    x0, x1 = rounds(x0, x1, r1)
    x0 = x0 + k0
    x1 = x1 + (k1 + jnp.uint32(3))
    x0, x1 = rounds(x0, x1, r2)
    x0 = x0 + k1
    x1 = x1 + (k2 + jnp.uint32(4))
    x0, x1 = rounds(x0, x1, r1)
    x0 = x0 + k2
    x1 = x1 + (k0 + jnp.uint32(5))
    return x0 ^ x1


def _sample_body(logits_ref, x_ref, p_ref, idx_ref, val_ref):
    step = pl.program_id(0)
    row = lax.broadcasted_iota(jnp.uint32, (B, CHUNK), 0)
    lane = lax.broadcasted_iota(jnp.uint32, (B, CHUNK), 1)
    kbase0 = row * jnp.uint32(D) + lane + jnp.uint32(42)
    col0 = lax.broadcasted_iota(jnp.int32, (B, CHUNK), 1)

    for dr in range(DRAWS):
        r = step * DRAWS + dr
        base = r.astype(jnp.uint32) * jnp.uint32(B * D)
        kbase = kbase0 + base

        def one_chunk(k, acc_m, acc_i):
            v = kbase + (k.astype(jnp.uint32) * jnp.uint32(CHUNK))
            bits = _threefry_bits_from_x1(v)
            f = lax.bitcast_convert_type(
                (bits >> jnp.uint32(9)) | jnp.uint32(0x3F800000), jnp.float32)
            f = f - np.float32(1.0)
            u = jnp.maximum(f * (np.float32(1.0) - TINY) + TINY, TINY)
            g = -jnp.log(-jnp.log(u))
            s = g + logits_ref[:, pl.ds(k * CHUNK, CHUNK)]
            upd = s > acc_m
            acc_m = jnp.where(upd, s, acc_m)
            acc_i = jnp.where(upd, col0 + k * CHUNK, acc_i)
            return acc_m, acc_i

        def chunk_body(h, carry):
            acc_m, acc_i = carry
            acc_m, acc_i = one_chunk(h * 2, acc_m, acc_i)
            acc_m, acc_i = one_chunk(h * 2 + 1, acc_m, acc_i)
            return acc_m, acc_i

        acc_m0 = jnp.full((B, CHUNK), -jnp.inf, jnp.float32)
        acc_i0 = jnp.zeros((B, CHUNK), jnp.int32)
        acc_m, acc_i = lax.fori_loop(
            0, NCHUNK // 2, chunk_body, (acc_m0, acc_i0))
        m = jnp.max(acc_m, axis=1, keepdims=True)
        idx = jnp.min(jnp.where(acc_m == m, acc_i, D), axis=1)
        idx_ref[dr] = idx.reshape(1, B).astype(jnp.int32)

    @pl.when(step == 0)
    def _():
        p = p_ref[...]
        e = jnp.exp(np.float32(RATE) * jnp.log(np.float32(1.0) - p))
        pre = np.float32(1.0) / (np.float32(1.0) - e)
        pre = jnp.where(jnp.isinf(pre), np.float32(1.0) / (np.float32(RATE) * p), pre)
        val_ref[...] = x_ref[...] * pre


_sample_call = pl.pallas_call(
    _sample_body,
    grid=(NSTEP,),
    in_specs=[
        pl.BlockSpec((B, D), lambda r: (0, 0)),
        pl.BlockSpec((B, D), lambda r: (0, 0)),
        pl.BlockSpec((B, D), lambda r: (0, 0)),
    ],
    out_specs=[
        pl.BlockSpec((DRAWS, 1, B), lambda r: (r, 0, 0)),
        pl.BlockSpec((B, D), lambda r: (0, 0)),
    ],
    out_shape=[
        jax.ShapeDtypeStruct((RATE, 1, B), jnp.int32),
        jax.ShapeDtypeStruct((B, D), jnp.float32),
    ],
    compiler_params=pltpu.CompilerParams(
        dimension_semantics=("parallel",),
    ),
)


def _make_scatter_kernel():
    info = plsc.get_sparse_core_info()
    nw = info.num_cores * info.num_subcores
    rows_per_w = B // nw

    @functools.partial(
        pl.kernel,
        mesh=plsc.VectorSubcoreMesh(core_axis_name="c", subcore_axis_name="s"),
        out_type=jax.ShapeDtypeStruct((B, D), jnp.float32),
        scratch_types=[
            pltpu.VMEM((IDXP,), jnp.int32),
            pltpu.VMEM((D,), jnp.float32),
            pltpu.VMEM((D,), jnp.float32),
        ],
        compiler_params=pltpu.CompilerParams(needs_layout_passes=False),
    )
    def _scatter_kernel(idx_hbm, val_hbm, out_hbm, idx_v, val_v, out_v):
        wid = lax.axis_index("s") * info.num_cores + lax.axis_index("c")
        zero = jnp.zeros((16,), jnp.float32)
        for t in range(rows_per_w):
            b = wid * rows_per_w + t
            pltpu.sync_copy(idx_hbm.at[b], idx_v)
            pltpu.sync_copy(val_hbm.at[b], val_v)

            def zbody(j, _):
                out_v[pl.ds(j * 16, 16)] = zero
                return _

            lax.fori_loop(0, D // 16, zbody, None)
            for c in range(CHUNKS):
                iv = idx_v[pl.ds(c * 16, 16)]
                vals = plsc.load_gather(val_v, [iv])
                plsc.store_scatter(out_v, [iv], vals)
            pltpu.sync_copy(out_v, out_hbm.at[b])

    return _scatter_kernel


def kernel(x):
    ha = jnp.abs(x)
    p = ha / (jnp.sum(ha, axis=1, keepdims=True) + 1e-10)
    logits = jnp.where(p > 0, jnp.log(p), -jnp.inf)
    idx, val = _sample_call(logits, x, p)
    idx_t = idx.reshape(RATE, B).T
    idx_pad = jnp.concatenate(
        [idx_t, jnp.broadcast_to(idx_t[:, -1:], (B, IDXP - RATE))], axis=1)
    return _make_scatter_kernel()(idx_pad, val)


# speedup vs baseline: 1.0573x; 1.0195x over previous
import functools

import jax
import jax.numpy as jnp
import numpy as np
from jax import lax
from jax.experimental import pallas as pl
from jax.experimental.pallas import tpu as pltpu
from jax.experimental.pallas import tpu_sc as plsc

B = 64
D = 8192
RATE = 819
CHUNKS = (RATE + 15) // 16
IDXP = CHUNKS * 16
TINY = np.float32(np.finfo(np.float32).tiny)


def _rotl(x, r):
    return (x << jnp.uint32(r)) | (x >> jnp.uint32(32 - r))


def _threefry_bits(c1):
    k0 = jnp.uint32(0)
    k1 = jnp.uint32(42)
    k2 = jnp.uint32(0 ^ 42 ^ 0x1BD11BDA)
    r1 = (13, 15, 26, 6)
    r2 = (17, 29, 16, 24)

    def rounds(x0, x1, rots):
        for r in rots:
            x0 = x0 + x1
            x1 = _rotl(x1, r)
            x1 = x1 ^ x0
        return x0, x1

    x0 = jnp.zeros_like(c1) + k0
    x1 = c1 + k1
    x0, x1 = rounds(x0, x1, r1)
    x0 = x0 + k1
    x1 = x1 + (k2 + jnp.uint32(1))
    x0, x1 = rounds(x0, x1, r2)
    x0 = x0 + k2
    x1 = x1 + (k0 + jnp.uint32(2))
    x0, x1 = rounds(x0, x1, r1)
    x0 = x0 + k0
    x1 = x1 + (k1 + jnp.uint32(3))
    x0, x1 = rounds(x0, x1, r2)
    x0 = x0 + k1
    x1 = x1 + (k2 + jnp.uint32(4))
    x0, x1 = rounds(x0, x1, r1)
    x0 = x0 + k2
    x1 = x1 + (k0 + jnp.uint32(5))
    return x0 ^ x1


CHUNK = 256
NCHUNK = D // CHUNK
DRAWS = 9
NSTEP = RATE // DRAWS


def _threefry_bits_from_x1(v):
    k1 = jnp.uint32(42)
    k2 = jnp.uint32(0 ^ 42 ^ 0x1BD11BDA)
    k0 = jnp.uint32(0)
    r1 = (13, 15, 26, 6)
    r2 = (17, 29, 16, 24)

    def rounds(x0, x1, rots):
        for r in rots:
            x0 = x0 + x1
            x1 = _rotl(x1, r)
            x1 = x1 ^ x0
        return x0, x1

    x0 = v
    x1 = _rotl(v, r1[0]) ^ x0
    x0, x1 = rounds(x0, x1, r1[1:])
    x0 = x0 + k1
    x1 = x1 + (k2 + jnp.uint32(1))
    x0, x1 = rounds(x0, x1, r2)
    x0 = x0 + k2
    x1 = x1 + (k0 + jnp.uint32(2))
    x0, x1 = rounds(x0, x1, r1)
    x0 = x0 + k0
    x1 = x1 + (k1 + jnp.uint32(3))
    x0, x1 = rounds(x0, x1, r2)
    x0 = x0 + k1
    x1 = x1 + (k2 + jnp.uint32(4))
    x0, x1 = rounds(x0, x1, r1)
    x0 = x0 + k2
    x1 = x1 + (k0 + jnp.uint32(5))
    return x0 ^ x1


def _sample_body(logits_ref, x_ref, p_ref, idx_ref, val_ref):
    step = pl.program_id(0)
    row = lax.broadcasted_iota(jnp.uint32, (B, CHUNK), 0)
    lane = lax.broadcasted_iota(jnp.uint32, (B, CHUNK), 1)
    kbase0 = row * jnp.uint32(D) + lane + jnp.uint32(42)
    col0 = lax.broadcasted_iota(jnp.int32, (B, CHUNK), 1)

    for dr in range(DRAWS):
        r = step * DRAWS + dr
        base = r.astype(jnp.uint32) * jnp.uint32(B * D)
        kbase = kbase0 + base

        def one_chunk(k, acc_m, acc_i):
            v = kbase + (k.astype(jnp.uint32) * jnp.uint32(CHUNK))
            bits = _threefry_bits_from_x1(v)
            f = lax.bitcast_convert_type(
                (bits >> jnp.uint32(9)) | jnp.uint32(0x3F800000), jnp.float32)
            f = f - np.float32(1.0)
            u = jnp.maximum(f * (np.float32(1.0) - TINY) + TINY, TINY)
            g = -jnp.log(-jnp.log(u))
            s = g + logits_ref[:, pl.ds(k * CHUNK, CHUNK)]
            upd = s > acc_m
            acc_m = jnp.where(upd, s, acc_m)
            acc_i = jnp.where(upd, col0 + k * CHUNK, acc_i)
            return acc_m, acc_i

        def chunk_body(h, carry):
            acc_m, acc_i = carry
            for j in range(4):
                acc_m, acc_i = one_chunk(h * 4 + j, acc_m, acc_i)
            return acc_m, acc_i

        acc_m0 = jnp.full((B, CHUNK), -jnp.inf, jnp.float32)
        acc_i0 = jnp.zeros((B, CHUNK), jnp.int32)
        acc_m, acc_i = lax.fori_loop(
            0, NCHUNK // 4, chunk_body, (acc_m0, acc_i0))
        m = jnp.max(acc_m, axis=1, keepdims=True)
        idx = jnp.min(jnp.where(acc_m == m, acc_i, D), axis=1)
        idx_ref[dr] = idx.reshape(1, B).astype(jnp.int32)

    @pl.when(step == 0)
    def _():
        p = p_ref[...]
        e = jnp.exp(np.float32(RATE) * jnp.log(np.float32(1.0) - p))
        pre = np.float32(1.0) / (np.float32(1.0) - e)
        pre = jnp.where(jnp.isinf(pre), np.float32(1.0) / (np.float32(RATE) * p), pre)
        val_ref[...] = x_ref[...] * pre


_sample_call = pl.pallas_call(
    _sample_body,
    grid=(NSTEP,),
    in_specs=[
        pl.BlockSpec((B, D), lambda r: (0, 0)),
        pl.BlockSpec((B, D), lambda r: (0, 0)),
        pl.BlockSpec((B, D), lambda r: (0, 0)),
    ],
    out_specs=[
        pl.BlockSpec((DRAWS, 1, B), lambda r: (r, 0, 0)),
        pl.BlockSpec((B, D), lambda r: (0, 0)),
    ],
    out_shape=[
        jax.ShapeDtypeStruct((RATE, 1, B), jnp.int32),
        jax.ShapeDtypeStruct((B, D), jnp.float32),
    ],
    compiler_params=pltpu.CompilerParams(
        dimension_semantics=("parallel",),
    ),
)


def _make_scatter_kernel():
    info = plsc.get_sparse_core_info()
    nw = info.num_cores * info.num_subcores
    rows_per_w = B // nw

    @functools.partial(
        pl.kernel,
        mesh=plsc.VectorSubcoreMesh(core_axis_name="c", subcore_axis_name="s"),
        out_type=jax.ShapeDtypeStruct((B, D), jnp.float32),
        scratch_types=[
            pltpu.VMEM((IDXP,), jnp.int32),
            pltpu.VMEM((D,), jnp.float32),
            pltpu.VMEM((D,), jnp.float32),
        ],
        compiler_params=pltpu.CompilerParams(needs_layout_passes=False),
    )
    def _scatter_kernel(idx_hbm, val_hbm, out_hbm, idx_v, val_v, out_v):
        wid = lax.axis_index("s") * info.num_cores + lax.axis_index("c")
        zero = jnp.zeros((16,), jnp.float32)
        for t in range(rows_per_w):
            b = wid * rows_per_w + t
            pltpu.sync_copy(idx_hbm.at[b], idx_v)
            pltpu.sync_copy(val_hbm.at[b], val_v)

            def zbody(j, _):
                out_v[pl.ds(j * 16, 16)] = zero
                return _

            lax.fori_loop(0, D // 16, zbody, None)
            for c in range(CHUNKS):
                iv = idx_v[pl.ds(c * 16, 16)]
                vals = plsc.load_gather(val_v, [iv])
                plsc.store_scatter(out_v, [iv], vals)
            pltpu.sync_copy(out_v, out_hbm.at[b])

    return _scatter_kernel


def kernel(x):
    ha = jnp.abs(x)
    p = ha / (jnp.sum(ha, axis=1, keepdims=True) + 1e-10)
    logits = jnp.where(p > 0, jnp.log(p), -jnp.inf)
    idx, val = _sample_call(logits, x, p)
    idx_t = idx.reshape(RATE, B).T
    idx_pad = jnp.concatenate(
        [idx_t, jnp.broadcast_to(idx_t[:, -1:], (B, IDXP - RATE))], axis=1)
    return _make_scatter_kernel()(idx_pad, val)
